# Initial kernel scaffold; baseline (speedup 1.0000x reference)
import functools

import jax
import jax.numpy as jnp
from jax import lax
from jax.experimental import pallas as pl
from jax.experimental.pallas import tpu as pltpu
from jax.experimental.pallas import tpu_sc as plsc

VOCAB = 100000
DIM = 64
B = 16384
CTX = 20
NEG = 20

NC = 2
NS = 16
NW = NC * NS
ROWS_PER_W = B // NW
CB = 32
NCHUNK = ROWS_PER_W // CB
GI = (CB * CTX) // 128
NLANE = DIM // 16


def _sc_logits_kernel():
    mesh = plsc.VectorSubcoreMesh(core_axis_name="c", subcore_axis_name="s")

    @functools.partial(
        pl.kernel,
        mesh=mesh,
        out_type=(
            jax.ShapeDtypeStruct((B,), jnp.float32),
            jax.ShapeDtypeStruct((B * NEG,), jnp.float32),
        ),
        scratch_types=[
            pltpu.VMEM((CB * CTX,), jnp.int32),
            pltpu.VMEM((CB,), jnp.int32),
            pltpu.VMEM((CB * NEG,), jnp.int32),
            pltpu.VMEM((CB * CTX, DIM), jnp.float32),
            pltpu.VMEM((CB, DIM), jnp.float32),
            pltpu.VMEM((CB * NEG, DIM), jnp.float32),
            pltpu.VMEM((CB,), jnp.float32),
            pltpu.VMEM((CB * NEG,), jnp.float32),
            pltpu.SemaphoreType.DMA,
        ],
    )
    def k(emb, oemb, ctx_idx_h, tgt_idx_h, neg_idx_h, pos_h, negl_h,
          ctx_i, tgt_i, neg_i, ctx_r, tgt_r, neg_r, pos_v, negl_v, sem):
        wid = lax.axis_index("s") * NC + lax.axis_index("c")

        def chunk_body(ci, carry):
            base = (wid * NCHUNK + ci) * CB
            pltpu.sync_copy(ctx_idx_h.at[pl.ds(base * CTX, CB * CTX)], ctx_i)
            pltpu.sync_copy(tgt_idx_h.at[pl.ds(base, CB)], tgt_i)
            pltpu.sync_copy(neg_idx_h.at[pl.ds(base * NEG, CB * NEG)], neg_i)
            copies = []
            for g in range(GI):
                copies.append(pltpu.async_copy(
                    emb.at[ctx_i.at[pl.ds(g * 128, 128)]],
                    ctx_r.at[pl.ds(g * 128, 128)], sem))
            for g in range(GI):
                copies.append(pltpu.async_copy(
                    oemb.at[neg_i.at[pl.ds(g * 128, 128)]],
                    neg_r.at[pl.ds(g * 128, 128)], sem))
            copies.append(pltpu.async_copy(oemb.at[tgt_i], tgt_r, sem))
            for cpy in copies:
                cpy.wait()

            def row_body(r, carry2):
                cvecs = []
                for d in range(NLANE):
                    a = ctx_r[r * CTX, pl.ds(d * 16, 16)]
                    for c in range(1, CTX):
                        a = a + ctx_r[r * CTX + c, pl.ds(d * 16, 16)]
                    cvecs.append(a * (1.0 / CTX))
                p = cvecs[0] * tgt_r[r, pl.ds(0, 16)]
                for d in range(1, NLANE):
                    p = p + cvecs[d] * tgt_r[r, pl.ds(d * 16, 16)]
                pos_v[r] = jnp.sum(p)
                for j in range(NEG):
                    q = cvecs[0] * neg_r[r * NEG + j, pl.ds(0, 16)]
                    for d in range(1, NLANE):
                        q = q + cvecs[d] * neg_r[r * NEG + j, pl.ds(d * 16, 16)]
                    negl_v[r * NEG + j] = jnp.sum(q)
                return carry2

            lax.fori_loop(0, CB, row_body, 0)
            pltpu.sync_copy(pos_v, pos_h.at[pl.ds(base, CB)])
            pltpu.sync_copy(negl_v, negl_h.at[pl.ds(base * NEG, CB * NEG)])
            return carry

        lax.fori_loop(0, NCHUNK, chunk_body, 0)

    return k


def _loss_tc(pos2d, neg2d):
    def body(pos_ref, neg_ref, out_ref):
        p = jax.nn.sigmoid(pos_ref[...])
        n = jax.nn.sigmoid(neg_ref[...])
        lp = jnp.sum(jnp.log(p + 1e-9))
        ln = jnp.sum(jnp.log(1.0 - n + 1e-9))
        out_ref[0, 0] = -(lp / B) - (ln / (B * NEG))

    return pl.pallas_call(
        body,
        out_shape=jax.ShapeDtypeStruct((1, 1), jnp.float32),
        out_specs=pl.BlockSpec(memory_space=pltpu.SMEM),
    )(pos2d, neg2d)


def kernel(context_words, target, negative_samples, embeddings,
           output_embeddings):
    ctx_flat = context_words.reshape(-1)
    neg_flat = negative_samples.reshape(-1)
    pos, negl = _sc_logits_kernel()(
        embeddings, output_embeddings, ctx_flat, target, neg_flat)
    loss = _loss_tc(pos.reshape(B // 128, 128),
                    negl.reshape(B * NEG // 128, 128))
    return loss[0, 0]


# speedup vs baseline: 9.5923x; 9.5923x over previous
import functools

import jax
import jax.numpy as jnp
from jax import lax
from jax.experimental import pallas as pl
from jax.experimental.pallas import tpu as pltpu
from jax.experimental.pallas import tpu_sc as plsc

VOCAB = 100000
DIM = 64
B = 16384
CTX = 20
NEG = 20

NC = 2
NS = 16
NW = NC * NS
ROWS_PER_W = B // NW
CB = 32
NCHUNK = ROWS_PER_W // CB
GI = (CB * CTX) // 128
NLANE = DIM // 16


def _sc_logits_kernel():
    mesh = plsc.VectorSubcoreMesh(core_axis_name="c", subcore_axis_name="s")

    @functools.partial(
        pl.kernel,
        mesh=mesh,
        compiler_params=pltpu.CompilerParams(
            needs_layout_passes=False, use_tc_tiling_on_sc=False),
        out_type=(
            jax.ShapeDtypeStruct((B,), jnp.float32),
            jax.ShapeDtypeStruct((B * NEG,), jnp.float32),
        ),
        scratch_types=[
            pltpu.VMEM((CB * CTX,), jnp.int32),
            pltpu.VMEM((CB,), jnp.int32),
            pltpu.VMEM((CB * NEG,), jnp.int32),
            pltpu.VMEM((CB * CTX, DIM), jnp.float32),
            pltpu.VMEM((CB, DIM), jnp.float32),
            pltpu.VMEM((CB * NEG, DIM), jnp.float32),
            pltpu.VMEM((CB,), jnp.float32),
            pltpu.VMEM((CB * NEG,), jnp.float32),
            pltpu.SemaphoreType.DMA,
        ],
    )
    def k(emb, oemb, ctx_idx_h, tgt_idx_h, neg_idx_h, pos_h, negl_h,
          ctx_i, tgt_i, neg_i, ctx_r, tgt_r, neg_r, pos_v, negl_v, sem):
        wid = lax.axis_index("s") * NC + lax.axis_index("c")

        def chunk_body(ci, carry):
            base = (wid * NCHUNK + ci) * CB
            pltpu.sync_copy(ctx_idx_h.at[pl.ds(base * CTX, CB * CTX)], ctx_i)
            pltpu.sync_copy(tgt_idx_h.at[pl.ds(base, CB)], tgt_i)
            pltpu.sync_copy(neg_idx_h.at[pl.ds(base * NEG, CB * NEG)], neg_i)
            copies = []
            for g in range(GI):
                copies.append(pltpu.async_copy(
                    emb.at[ctx_i.at[pl.ds(g * 128, 128)]],
                    ctx_r.at[pl.ds(g * 128, 128)], sem))
            for g in range(GI):
                copies.append(pltpu.async_copy(
                    oemb.at[neg_i.at[pl.ds(g * 128, 128)]],
                    neg_r.at[pl.ds(g * 128, 128)], sem))
            copies.append(pltpu.async_copy(oemb.at[tgt_i], tgt_r, sem))
            for cpy in copies:
                cpy.wait()

            lane = lax.iota(jnp.int32, 16)
            for g in range(CB // 16):
                rb = g * 16
                zero = jnp.zeros((16,), jnp.float32)

                def r_body(r16, accs):
                    r = rb + r16
                    mask = lane == r16
                    cvecs = []
                    for d in range(NLANE):
                        a = ctx_r[r * CTX, pl.ds(d * 16, 16)]
                        for c in range(1, CTX):
                            a = a + ctx_r[r * CTX + c, pl.ds(d * 16, 16)]
                        cvecs.append(a * (1.0 / CTX))

                    def dot_bcast(ref, row):
                        acc = cvecs[0] * ref[row, pl.ds(0, 16)]
                        for d in range(1, NLANE):
                            acc = acc + cvecs[d] * ref[row, pl.ds(d * 16, 16)]
                        return jnp.full((16,), jnp.sum(acc), jnp.float32)

                    out = [jnp.where(mask, dot_bcast(tgt_r, r), accs[0])]
                    for j in range(NEG):
                        out.append(jnp.where(
                            mask, dot_bcast(neg_r, r * NEG + j), accs[1 + j]))
                    return tuple(out)

                accs = lax.fori_loop(0, 16, r_body, (zero,) * (1 + NEG))
                pos_v[pl.ds(rb, 16)] = accs[0]
                for j in range(NEG):
                    negl_v[pl.ds(rb * NEG + j * 16, 16)] = accs[1 + j]
            pltpu.sync_copy(pos_v, pos_h.at[pl.ds(base, CB)])
            pltpu.sync_copy(negl_v, negl_h.at[pl.ds(base * NEG, CB * NEG)])
            return carry

        lax.fori_loop(0, NCHUNK, chunk_body, 0)

    return k


def _loss_tc(pos2d, neg2d):
    def body(pos_ref, neg_ref, out_ref):
        p = jax.nn.sigmoid(pos_ref[...])
        n = jax.nn.sigmoid(neg_ref[...])
        lp = jnp.sum(jnp.log(p + 1e-9))
        ln = jnp.sum(jnp.log(1.0 - n + 1e-9))
        out_ref[0, 0] = -(lp / B) - (ln / (B * NEG))

    return pl.pallas_call(
        body,
        out_shape=jax.ShapeDtypeStruct((1, 1), jnp.float32),
        out_specs=pl.BlockSpec(memory_space=pltpu.SMEM),
    )(pos2d, neg2d)


def kernel(context_words, target, negative_samples, embeddings,
           output_embeddings):
    ctx_flat = context_words.reshape(-1)
    neg_flat = negative_samples.reshape(-1)
    pos, negl = _sc_logits_kernel()(
        embeddings, output_embeddings, ctx_flat, target, neg_flat)
    loss = _loss_tc(pos.reshape(B // 128, 128),
                    negl.reshape(B * NEG // 128, 128))
    return loss[0, 0]
